# Initial kernel scaffold; baseline (speedup 1.0000x reference)
import jax
import jax.numpy as jnp
from jax.experimental import pallas as pl


def kernel(x_nchw, conv1_w, conv1_b, conv2_w, conv2_b, conv3_w, conv3_b, conv4_w, conv4_b, fc1_w, fc1_b, fc2_w, fc2_b):
    raise NotImplementedError("write your pallas kernel here")


# speedup vs baseline: 3.2050x; 3.2050x over previous
import jax
import jax.numpy as jnp
from jax.experimental import pallas as pl
from jax.experimental.pallas import tpu as pltpu

_B = 32


def _band_mats(w, W, Cout):
    Cin = w.shape[0]
    OW = W - 2
    w4 = w.reshape(Cin, 3, 3, Cout)
    j = jnp.arange(OW)
    mats = []
    for di in range(3):
        M = jnp.zeros((W, Cin, OW, Cout), w.dtype)
        for dj in range(3):
            M = M.at[j + dj, :, j, :].add(w4[:, di, dj, :])
        mats.append(M.reshape(W * Cin, OW * Cout))
    return mats


def _bconv(x, m0, m1, m2, brow):
    B, H, WC = x.shape
    OH = H - 2
    OWC = m0.shape[1]
    acc = jnp.dot(x[:, 0:OH, :].reshape(B * OH, WC), m0,
                  preferred_element_type=jnp.float32)
    acc = acc + jnp.dot(x[:, 1:1 + OH, :].reshape(B * OH, WC), m1,
                        preferred_element_type=jnp.float32)
    acc = acc + jnp.dot(x[:, 2:2 + OH, :].reshape(B * OH, WC), m2,
                        preferred_element_type=jnp.float32)
    out = jnp.maximum(acc + brow, 0.0).astype(jnp.bfloat16)
    return out.reshape(B, OH, OWC)


def _bpool(x, C):
    B, H, WC = x.shape
    W = WC // C
    v = x.reshape(B, H // 2, 2, WC)
    m = jnp.maximum(v[:, :, 0, :], v[:, :, 1, :])
    m = m.reshape(B, H // 2, W // 2, 2, C)
    m = jnp.maximum(m[:, :, :, 0, :], m[:, :, :, 1, :])
    return m.reshape(B, H // 2, (W // 2) * C)


def _ffnet_kernel(x_ref,
                  a0_ref, a1_ref, a2_ref, ab_ref,
                  b0_ref, b1_ref, b2_ref, bb_ref,
                  c0_ref, c1_ref, c2_ref, cb_ref,
                  d0_ref, d1_ref, d2_ref, db_ref,
                  f1w_ref, f1b_ref, f2w_ref, f2b_ref, o_ref):
    B = x_ref.shape[0]
    a = _bconv(x_ref[...], a0_ref[...], a1_ref[...], a2_ref[...], ab_ref[...])
    a = _bconv(a, b0_ref[...], b1_ref[...], b2_ref[...], bb_ref[...])
    a = _bpool(a, 32)
    a = _bconv(a, c0_ref[...], c1_ref[...], c2_ref[...], cb_ref[...])
    a = _bconv(a, d0_ref[...], d1_ref[...], d2_ref[...], db_ref[...])
    a = _bpool(a, 64)
    h = jnp.dot(a.reshape(B, 1600), f1w_ref[...],
                preferred_element_type=jnp.float32)
    h = jnp.maximum(h + f1b_ref[...], 0.0).astype(jnp.bfloat16)
    y = jnp.dot(h, f2w_ref[...], preferred_element_type=jnp.float32)
    o_ref[...] = y + f2b_ref[...]


@jax.jit
def _run(x_nchw, c1w, c1b, c2w, c2b, c3w, c3b, c4w, c4b,
         f1w, f1b, f2w, f2b):
    N = x_nchw.shape[0]
    B = _B
    while N % B:
        B //= 2
    x = jnp.transpose(x_nchw, (0, 2, 3, 1)).astype(jnp.bfloat16)
    x = x.reshape(N, 32, 32 * 3)

    ws = []
    for w, b, W in ((c1w, c1b, 32), (c2w, c2b, 30),
                    (c3w, c3b, 14), (c4w, c4b, 12)):
        Cout = b.shape[1]
        ws += _band_mats(w, W, Cout)
        ws.append(jnp.tile(b, (1, W - 2)))
    ws += [f1w, f1b, f2w, f2b]

    def _full(w):
        return pl.BlockSpec(w.shape, lambda i, n=w.ndim: (0,) * n)

    out = pl.pallas_call(
        _ffnet_kernel,
        out_shape=jax.ShapeDtypeStruct((N, 128), jnp.float32),
        grid=(N // B,),
        in_specs=[pl.BlockSpec((B, 32, 32 * 3), lambda i: (i, 0, 0))]
        + [_full(w) for w in ws],
        out_specs=pl.BlockSpec((B, 128), lambda i: (i, 0)),
        compiler_params=pltpu.CompilerParams(
            dimension_semantics=("parallel",)),
    )(x, *ws)
    return out[:, :10]


def kernel(x_nchw, conv1_w, conv1_b, conv2_w, conv2_b, conv3_w, conv3_b,
           conv4_w, conv4_b, fc1_w, fc1_b, fc2_w, fc2_b):
    return _run(x_nchw, conv1_w, conv1_b, conv2_w, conv2_b, conv3_w, conv3_b,
                conv4_w, conv4_b, fc1_w, fc1_b, fc2_w, fc2_b)


# speedup vs baseline: 8.3065x; 2.5917x over previous
import jax
import jax.numpy as jnp
from jax.experimental import pallas as pl
from jax.experimental.pallas import tpu as pltpu

_B = 32


def _band_mats(w, W, Cout, pool):
    Cin = w.shape[0]
    OW = W - 2
    w4 = w.reshape(Cin, 3, 3, Cout)
    j = jnp.arange(OW)
    mats = []
    for di in range(3):
        M = jnp.zeros((W, Cin, OW, Cout), w.dtype)
        for dj in range(3):
            M = M.at[j + dj, :, j, :].add(w4[:, di, dj, :])
        if pool:
            M = M.reshape(W * Cin, OW // 2, 2, Cout)
            M = jnp.transpose(M, (0, 2, 1, 3))
        mats.append(M.reshape(W * Cin, OW * Cout))
    return mats


def _bconv(x, m0, m1, m2, brow):
    B, H, WC = x.shape
    OH = H - 2
    OWC = m0.shape[1]
    acc = jnp.dot(x[:, 0:OH, :].reshape(B * OH, WC), m0,
                  preferred_element_type=jnp.float32)
    acc = acc + jnp.dot(x[:, 1:1 + OH, :].reshape(B * OH, WC), m1,
                        preferred_element_type=jnp.float32)
    acc = acc + jnp.dot(x[:, 2:2 + OH, :].reshape(B * OH, WC), m2,
                        preferred_element_type=jnp.float32)
    out = jnp.maximum(acc + brow, 0.0).astype(jnp.bfloat16)
    return out.reshape(B, OH, OWC)


def _bpool(x):
    B, H, WC = x.shape
    half = WC // 2
    v = x.reshape(B, H // 2, 2, WC)
    m = jnp.maximum(v[:, :, 0, :], v[:, :, 1, :])
    return jnp.maximum(m[:, :, :half], m[:, :, half:])


def _ffnet_kernel(x_ref,
                  a0_ref, a1_ref, a2_ref, ab_ref,
                  b0_ref, b1_ref, b2_ref, bb_ref,
                  c0_ref, c1_ref, c2_ref, cb_ref,
                  d0_ref, d1_ref, d2_ref, db_ref,
                  f1w_ref, f1b_ref, f2w_ref, f2b_ref, o_ref):
    B = x_ref.shape[0]
    a = _bconv(x_ref[...], a0_ref[...], a1_ref[...], a2_ref[...], ab_ref[...])
    a = _bconv(a, b0_ref[...], b1_ref[...], b2_ref[...], bb_ref[...])
    a = _bpool(a)
    a = _bconv(a, c0_ref[...], c1_ref[...], c2_ref[...], cb_ref[...])
    a = _bconv(a, d0_ref[...], d1_ref[...], d2_ref[...], db_ref[...])
    a = _bpool(a)
    h = jnp.dot(a.reshape(B, 1600), f1w_ref[...],
                preferred_element_type=jnp.float32)
    h = jnp.maximum(h + f1b_ref[...], 0.0).astype(jnp.bfloat16)
    y = jnp.dot(h, f2w_ref[...], preferred_element_type=jnp.float32)
    o_ref[...] = y + f2b_ref[...]


@jax.jit
def _run(x_nchw, c1w, c1b, c2w, c2b, c3w, c3b, c4w, c4b,
         f1w, f1b, f2w, f2b):
    N = x_nchw.shape[0]
    B = _B
    while N % B:
        B //= 2
    x = jnp.transpose(x_nchw, (0, 2, 3, 1)).astype(jnp.bfloat16)
    x = x.reshape(N, 32, 32 * 3)

    ws = []
    for w, b, W, pool in ((c1w, c1b, 32, False), (c2w, c2b, 30, True),
                          (c3w, c3b, 14, False), (c4w, c4b, 12, True)):
        Cout = b.shape[1]
        ws += _band_mats(w, W, Cout, pool)
        ws.append(jnp.tile(b, (1, W - 2)))
    ws += [f1w, f1b, f2w, f2b]

    def _full(w):
        return pl.BlockSpec(w.shape, lambda i, n=w.ndim: (0,) * n)

    out = pl.pallas_call(
        _ffnet_kernel,
        out_shape=jax.ShapeDtypeStruct((N, 128), jnp.float32),
        grid=(N // B,),
        in_specs=[pl.BlockSpec((B, 32, 32 * 3), lambda i: (i, 0, 0))]
        + [_full(w) for w in ws],
        out_specs=pl.BlockSpec((B, 128), lambda i: (i, 0)),
        compiler_params=pltpu.CompilerParams(
            dimension_semantics=("parallel",)),
    )(x, *ws)
    return out[:, :10]


def kernel(x_nchw, conv1_w, conv1_b, conv2_w, conv2_b, conv3_w, conv3_b,
           conv4_w, conv4_b, fc1_w, fc1_b, fc2_w, fc2_b):
    return _run(x_nchw, conv1_w, conv1_b, conv2_w, conv2_b, conv3_w, conv3_b,
                conv4_w, conv4_b, fc1_w, fc1_b, fc2_w, fc2_b)
